# baseline (speedup 1.0000x reference)
import functools

import jax
import jax.numpy as jnp
from jax import lax
from jax.experimental import pallas as pl
from jax.experimental.pallas import tpu as pltpu
from jax.experimental.pallas import tpu_sc as plsc

N_EXPERTS = 64
N_GROUPS = 8
GROUP_SIZE = N_EXPERTS // N_GROUPS
TOPK_GROUPS = 4
TOPK = 8
SCALING_FACTOR = 2.5


def _scores_body(h_ref, w_ref, rb_ref, sb_ref, s_ref):
    h = h_ref[...]
    w = w_ref[...]
    logits = jax.lax.dot_general(
        w, h, (((1,), (1,)), ((), ())), preferred_element_type=jnp.float32
    )
    logits = logits + rb_ref[...]
    s_ref[...] = jax.nn.sigmoid(logits) + sb_ref[...]


def _scores_tc(hidden_tensor, weight, router_bias, scores_bias):
    t, d = hidden_tensor.shape
    bt = 4096 if t % 4096 == 0 else t
    rb = router_bias.reshape(N_EXPERTS, 1)
    sb = scores_bias.reshape(N_EXPERTS, 1)
    return pl.pallas_call(
        _scores_body,
        grid=(t // bt,),
        in_specs=[
            pl.BlockSpec((bt, d), lambda i: (i, 0)),
            pl.BlockSpec((N_EXPERTS, d), lambda i: (0, 0)),
            pl.BlockSpec((N_EXPERTS, 1), lambda i: (0, 0)),
            pl.BlockSpec((N_EXPERTS, 1), lambda i: (0, 0)),
        ],
        out_specs=pl.BlockSpec((N_EXPERTS, bt), lambda i: (0, i)),
        out_shape=jax.ShapeDtypeStruct((N_EXPERTS, t), jnp.float32),
        compiler_params=pltpu.CompilerParams(
            dimension_semantics=("arbitrary",),
        ),
    )(hidden_tensor, weight, rb, sb)


def _make_sc_router(t):
    info = plsc.get_sparse_core_info()
    nc, ns, nl = info.num_cores, info.num_subcores, info.num_lanes
    nw = nc * ns
    tpw = t // nw
    nblk = tpw // nl
    mesh = plsc.VectorSubcoreMesh(core_axis_name="c", subcore_axis_name="s")

    @functools.partial(
        pl.kernel,
        mesh=mesh,
        out_type=[
            jax.ShapeDtypeStruct((TOPK * t,), jnp.int32),
            jax.ShapeDtypeStruct((TOPK * t,), jnp.float32),
        ],
        scratch_types=[
            pltpu.VMEM((N_EXPERTS * tpw,), jnp.float32),
            pltpu.VMEM((TOPK * tpw,), jnp.int32),
            pltpu.VMEM((TOPK * tpw,), jnp.float32),
            pltpu.SemaphoreType.DMA,
        ],
    )
    def sc_router(s_hbm, idx_hbm, wts_hbm, sbuf, idxbuf, wtsbuf, sem):
        wid = lax.axis_index("s") * nc + lax.axis_index("c")
        base = wid * tpw
        descs = [
            pltpu.async_copy(
                s_hbm.at[e, pl.ds(base, tpw)],
                sbuf.at[pl.ds(e * tpw, tpw)],
                sem,
            )
            for e in range(N_EXPERTS)
        ]
        for dsc in descs:
            dsc.wait()
        neg1 = jnp.full((nl,), -1.0, jnp.float32)

        def block(j, carry):
            off = j * nl
            sc = [
                sbuf[pl.ds(e * tpw + off, nl)] for e in range(N_EXPERTS)
            ]
            gw = []
            for g in range(N_GROUPS):
                m1 = sc[g * GROUP_SIZE]
                m2 = neg1
                for k in range(1, GROUP_SIZE):
                    v = sc[g * GROUP_SIZE + k]
                    gt = v > m1
                    m2 = jnp.where(gt, m1, jnp.maximum(m2, v))
                    m1 = jnp.where(gt, v, m1)
                gw.append(m1 + m2)
            zero = jnp.zeros((nl,), jnp.float32)
            self_f = [zero for _ in range(N_GROUPS)]
            for _ in range(TOPK_GROUPS):
                cand = [
                    gw[g] - self_f[g] * 4.0 for g in range(N_GROUPS)
                ]
                m = cand[0]
                for g in range(1, N_GROUPS):
                    m = jnp.maximum(m, cand[g])
                takenf = zero
                for g in range(N_GROUPS):
                    takef = jnp.where(cand[g] == m, 1.0 - takenf, 0.0)
                    self_f[g] = self_f[g] + takef
                    takenf = jnp.maximum(takenf, takef)
            for e in range(N_EXPERTS):
                sc[e] = jnp.where(
                    self_f[e // GROUP_SIZE] == 1.0, sc[e], -1.0
                )
            vals, eids = [], []
            for _ in range(TOPK):
                m = sc[0]
                for e in range(1, N_EXPERTS):
                    m = jnp.maximum(m, sc[e])
                eidf = zero
                foundf = zero
                for e in range(N_EXPERTS):
                    takef = jnp.where(sc[e] == m, 1.0 - foundf, 0.0)
                    eidf = eidf + takef * float(e)
                    sc[e] = sc[e] - takef * 2.0
                    foundf = jnp.maximum(foundf, takef)
                vals.append(m)
                eids.append(eidf.astype(jnp.int32))
            wsum = vals[0]
            for r in range(1, TOPK):
                wsum = wsum + vals[r]
            wsum = wsum + 1e-20
            for r in range(TOPK):
                idxbuf[pl.ds(r * tpw + off, nl)] = eids[r]
                wtsbuf[pl.ds(r * tpw + off, nl)] = (
                    vals[r] / wsum * SCALING_FACTOR
                )
            return carry

        lax.fori_loop(0, nblk, block, 0)
        for r in range(TOPK):
            pltpu.sync_copy(
                idxbuf.at[pl.ds(r * tpw, tpw)],
                idx_hbm.at[pl.ds(r * t + base, tpw)],
            )
            pltpu.sync_copy(
                wtsbuf.at[pl.ds(r * tpw, tpw)],
                wts_hbm.at[pl.ds(r * t + base, tpw)],
            )

    return sc_router


def kernel(hidden_tensor, weight, router_bias, scores_bias):
    t, _ = hidden_tensor.shape
    s = _scores_tc(hidden_tensor, weight, router_bias, scores_bias)
    idx_f, wts_f = _make_sc_router(t)(s)
    return (
        idx_f.reshape(TOPK, t).T,
        wts_f.reshape(TOPK, t).T,
    )


# speedup vs baseline: 7.1079x; 7.1079x over previous
import jax
import jax.numpy as jnp
from jax.experimental import pallas as pl
from jax.experimental.pallas import tpu as pltpu

N_EXPERTS = 64
N_GROUPS = 8
GROUP_SIZE = N_EXPERTS // N_GROUPS
TOPK_GROUPS = 4
TOPK = 8
SCALING_FACTOR = 2.5


def _router_body(h_ref, w_ref, rb_ref, sb_ref, idx_ref, wts_ref):
    bt = h_ref.shape[0]
    h = h_ref[...]
    w = w_ref[...]
    logits = jax.lax.dot_general(
        w, h, (((1,), (1,)), ((), ())), preferred_element_type=jnp.float32
    )
    logits = logits + rb_ref[...]
    s = jax.nn.sigmoid(logits) + sb_ref[...]

    m = jnp.maximum(s[0:32, :], s[32:64, :])
    m2 = jnp.minimum(s[0:32, :], s[32:64, :])
    for half in (16, 8):
        a, b = m[:half, :], m[half : 2 * half, :]
        sa, sb = m2[:half, :], m2[half : 2 * half, :]
        m2 = jnp.maximum(jnp.minimum(a, b), jnp.maximum(sa, sb))
        m = jnp.maximum(a, b)
    gw = m + m2

    ri8 = jax.lax.broadcasted_iota(jnp.int32, (N_GROUPS, bt), 0)
    selmask8 = jnp.zeros((N_GROUPS, bt), dtype=jnp.bool_)
    for _ in range(TOPK_GROUPS):
        mg = jnp.max(gw, axis=0, keepdims=True)
        gidx = jnp.min(
            jnp.where(gw == mg, ri8, N_GROUPS), axis=0, keepdims=True
        )
        eq = ri8 == gidx
        selmask8 = jnp.logical_or(selmask8, eq)
        gw = jnp.where(eq, -1.0, gw)

    bigmask = jnp.concatenate([selmask8] * N_GROUPS, axis=0)
    masked = jnp.where(bigmask, s, 0.0)

    ri64 = jax.lax.broadcasted_iota(jnp.int32, (N_EXPERTS, bt), 0)
    eid = (ri64 & 7) * 8 + (ri64 >> 3)
    idx_rows, val_rows = [], []
    for _ in range(TOPK):
        mv = jnp.max(masked, axis=0, keepdims=True)
        idx = jnp.min(
            jnp.where(masked == mv, eid, N_EXPERTS), axis=0, keepdims=True
        )
        idx_rows.append(idx)
        val_rows.append(mv)
        masked = jnp.where(eid == idx, -1.0, masked)
    vals = jnp.concatenate(val_rows, axis=0)
    idxs = jnp.concatenate(idx_rows, axis=0)
    wsum = jnp.sum(vals, axis=0, keepdims=True) + 1e-20
    idx_ref[...] = idxs
    wts_ref[...] = vals / wsum * SCALING_FACTOR


def kernel(hidden_tensor, weight, router_bias, scores_bias):
    t, d = hidden_tensor.shape
    bt = 4096 if t % 4096 == 0 else t
    grid = t // bt
    perm = jnp.arange(N_EXPERTS).reshape(N_GROUPS, GROUP_SIZE).T.reshape(-1)
    w_p = weight[perm]
    rb = router_bias[perm].reshape(N_EXPERTS, 1)
    sb = scores_bias[perm].reshape(N_EXPERTS, 1)
    idx_t, wts_t = pl.pallas_call(
        _router_body,
        grid=(grid,),
        in_specs=[
            pl.BlockSpec((bt, d), lambda i: (i, 0)),
            pl.BlockSpec((N_EXPERTS, d), lambda i: (0, 0)),
            pl.BlockSpec((N_EXPERTS, 1), lambda i: (0, 0)),
            pl.BlockSpec((N_EXPERTS, 1), lambda i: (0, 0)),
        ],
        out_specs=[
            pl.BlockSpec((TOPK, bt), lambda i: (0, i)),
            pl.BlockSpec((TOPK, bt), lambda i: (0, i)),
        ],
        out_shape=[
            jax.ShapeDtypeStruct((TOPK, t), jnp.int32),
            jax.ShapeDtypeStruct((TOPK, t), jnp.float32),
        ],
        compiler_params=pltpu.CompilerParams(
            dimension_semantics=("parallel",),
        ),
    )(hidden_tensor, w_p, rb, sb)
    return idx_t.T, wts_t.T
